# baseline (speedup 1.0000x reference)
import functools

import jax
import jax.numpy as jnp
from jax import lax
from jax.experimental import pallas as pl
from jax.experimental.pallas import tpu as pltpu
from jax.experimental.pallas import tpu_sc as plsc


def kernel(x, pos_table):
    seqlen = x.shape[1]
    _, dim = pos_table.shape

    info = plsc.get_sparse_core_info()
    nc, ns = info.num_cores, info.num_subcores
    nw = nc * ns
    assert seqlen % nw == 0
    rows_per_w = seqlen // nw

    mesh = plsc.VectorSubcoreMesh(core_axis_name="c", subcore_axis_name="s")

    @functools.partial(
        pl.kernel,
        mesh=mesh,
        out_type=jax.ShapeDtypeStruct((seqlen, dim), pos_table.dtype),
    )
    def copy_k(table_hbm, out_hbm):
        wid = lax.axis_index("s") * nc + lax.axis_index("c")
        base = wid * rows_per_w
        pltpu.sync_copy(
            table_hbm.at[pl.ds(base, rows_per_w)],
            out_hbm.at[pl.ds(base, rows_per_w)],
        )

    return copy_k(pos_table)


# speedup vs baseline: 24.1689x; 24.1689x over previous
import functools

import jax
import jax.numpy as jnp
from jax import lax
from jax.experimental import pallas as pl
from jax.experimental.pallas import tpu as pltpu
from jax.experimental.pallas import tpu_sc as plsc

_CHUNK_ROWS = 32


def kernel(x, pos_table):
    seqlen = x.shape[1]
    _, dim = pos_table.shape

    info = plsc.get_sparse_core_info()
    nc, ns = info.num_cores, info.num_subcores
    nw = nc * ns
    assert seqlen % nw == 0
    rows_per_w = seqlen // nw
    ch = min(_CHUNK_ROWS, rows_per_w)
    assert rows_per_w % ch == 0
    nch = rows_per_w // ch

    mesh = plsc.VectorSubcoreMesh(core_axis_name="c", subcore_axis_name="s")

    @functools.partial(
        pl.kernel,
        mesh=mesh,
        out_type=jax.ShapeDtypeStruct((seqlen, dim), pos_table.dtype),
        scratch_types=[
            pltpu.VMEM((ch, dim), pos_table.dtype),
            pltpu.VMEM((ch, dim), pos_table.dtype),
            pltpu.SemaphoreType.DMA,
            pltpu.SemaphoreType.DMA,
            pltpu.SemaphoreType.DMA,
            pltpu.SemaphoreType.DMA,
        ],
    )
    def copy_k(table_hbm, out_hbm, buf0, buf1, isem0, isem1, osem0, osem1):
        wid = lax.axis_index("s") * nc + lax.axis_index("c")
        base = wid * rows_per_w
        bufs = (buf0, buf1)
        isems = (isem0, isem1)
        osems = (osem0, osem1)

        in_h = [None] * nch
        out_h = [None] * nch
        for i in range(min(2, nch)):
            in_h[i] = pltpu.async_copy(
                table_hbm.at[pl.ds(base + i * ch, ch)], bufs[i % 2], isems[i % 2]
            )
        for i in range(nch):
            b = i % 2
            in_h[i].wait()
            out_h[i] = pltpu.async_copy(
                bufs[b], out_hbm.at[pl.ds(base + i * ch, ch)], osems[b]
            )
            if i + 2 < nch:
                out_h[i].wait()
                in_h[i + 2] = pltpu.async_copy(
                    table_hbm.at[pl.ds(base + (i + 2) * ch, ch)], bufs[b], isems[b]
                )
        for i in range(max(0, nch - 2), nch):
            out_h[i].wait()

    return copy_k(pos_table)


# speedup vs baseline: 24.7781x; 1.0252x over previous
import functools

import jax
import jax.numpy as jnp
from jax import lax
from jax.experimental import pallas as pl
from jax.experimental.pallas import tpu as pltpu
from jax.experimental.pallas import tpu_sc as plsc

_CHUNK_ROWS = 32
_NBUF = 3


def kernel(x, pos_table):
    seqlen = x.shape[1]
    _, dim = pos_table.shape

    info = plsc.get_sparse_core_info()
    nc, ns = info.num_cores, info.num_subcores
    nw = nc * ns
    assert seqlen % nw == 0
    rows_per_w = seqlen // nw
    ch = min(_CHUNK_ROWS, rows_per_w)
    assert rows_per_w % ch == 0
    nch = rows_per_w // ch
    nbuf = min(_NBUF, nch)

    mesh = plsc.VectorSubcoreMesh(core_axis_name="c", subcore_axis_name="s")

    @functools.partial(
        pl.kernel,
        mesh=mesh,
        out_type=jax.ShapeDtypeStruct((seqlen, dim), pos_table.dtype),
        scratch_types=(
            [pltpu.VMEM((ch, dim), pos_table.dtype)] * nbuf
            + [pltpu.SemaphoreType.DMA] * (2 * nbuf)
        ),
    )
    def copy_k(table_hbm, out_hbm, *scratch):
        bufs = scratch[:nbuf]
        isems = scratch[nbuf : 2 * nbuf]
        osems = scratch[2 * nbuf :]
        wid = lax.axis_index("s") * nc + lax.axis_index("c")
        base = wid * rows_per_w

        in_h = [None] * nch
        out_h = [None] * nch
        for i in range(nbuf):
            in_h[i] = pltpu.async_copy(
                table_hbm.at[pl.ds(base + i * ch, ch)], bufs[i % nbuf], isems[i % nbuf]
            )
        for i in range(nch):
            b = i % nbuf
            in_h[i].wait()
            out_h[i] = pltpu.async_copy(
                bufs[b], out_hbm.at[pl.ds(base + i * ch, ch)], osems[b]
            )
            if i + nbuf < nch:
                out_h[i].wait()
                in_h[i + nbuf] = pltpu.async_copy(
                    table_hbm.at[pl.ds(base + (i + nbuf) * ch, ch)], bufs[b], isems[b]
                )
        for i in range(max(0, nch - nbuf), nch):
            out_h[i].wait()

    return copy_k(pos_table)
